# Initial kernel scaffold; baseline (speedup 1.0000x reference)
import jax
import jax.numpy as jnp
from jax.experimental import pallas as pl


def kernel(positions, position_bias):
    raise NotImplementedError("write your pallas kernel here")


# speedup vs baseline: 285.4409x; 285.4409x over previous
import jax
import jax.numpy as jnp
from jax import lax
from jax.experimental import pallas as pl
from jax.experimental.pallas import tpu as pltpu
from jax.experimental.pallas import tpu_sc as plsc

_BATCH = 16384
_HIST = 200
_N = _BATCH * _HIST
_NC = 2
_NS = 16
_NW = _NC * _NS
_PER_W = _N // _NW
_CHUNK = 20480
_NCHUNK = _PER_W // _CHUNK
_L = 16
_TAB = 200


def _body(pos_hbm, tab_hbm, out_hbm, tab_v, idx_v, out_v):
    wid = lax.axis_index("s") * _NC + lax.axis_index("c")
    base = wid * _PER_W
    pltpu.sync_copy(tab_hbm, tab_v)
    for ci in range(_NCHUNK):
        off = base + ci * _CHUNK
        pltpu.sync_copy(pos_hbm.at[pl.ds(off, _CHUNK)], idx_v)

        @plsc.parallel_loop(0, _CHUNK // _L, unroll=8)
        def _gather(i):
            s = pl.ds(i * _L, _L)
            out_v[s] = plsc.load_gather(tab_v, [idx_v[s]])

        pltpu.sync_copy(out_v, out_hbm.at[pl.ds(off, _CHUNK)])


def kernel(positions, position_bias):
    pos_flat = positions.reshape(-1).astype(jnp.int32)
    tab = position_bias.reshape(-1)
    mesh = plsc.VectorSubcoreMesh(core_axis_name="c", subcore_axis_name="s")
    out = pl.kernel(
        _body,
        out_type=jax.ShapeDtypeStruct((_N,), jnp.float32),
        mesh=mesh,
        compiler_params=pltpu.CompilerParams(needs_layout_passes=False),
        scratch_types=[
            pltpu.VMEM((_TAB,), jnp.float32),
            pltpu.VMEM((_CHUNK,), jnp.int32),
            pltpu.VMEM((_CHUNK,), jnp.float32),
        ],
    )(pos_flat, tab)
    return out.reshape(_BATCH, _HIST)


# speedup vs baseline: 482.3402x; 1.6898x over previous
import jax
import jax.numpy as jnp
from jax import lax
from jax.experimental import pallas as pl
from jax.experimental.pallas import tpu as pltpu
from jax.experimental.pallas import tpu_sc as plsc

_BATCH = 16384
_HIST = 200
_NC = 2
_NS = 16
_NW = _NC * _NS
_ROWS_W = _BATCH // _NW
_RCHUNK = 128
_NRC = _ROWS_W // _RCHUNK
_L = 16
_TAB = 200
_OFFS = tuple(j * _L for j in range(12)) + (184,)


def _body(pos_hbm, tab_hbm, out_hbm, tab_v, idx_v, out_v):
    wid = lax.axis_index("s") * _NC + lax.axis_index("c")
    row0 = wid * _ROWS_W
    pltpu.sync_copy(tab_hbm, tab_v)
    for ci in range(_NRC):
        r0 = row0 + ci * _RCHUNK
        pltpu.sync_copy(pos_hbm.at[pl.ds(r0, _RCHUNK)], idx_v)

        @plsc.parallel_loop(0, _RCHUNK)
        def _gather(r):
            for off in _OFFS:
                s = pl.ds(off, _L)
                out_v[r, s] = plsc.load_gather(tab_v, [idx_v[r, s]])

        pltpu.sync_copy(out_v, out_hbm.at[pl.ds(r0, _RCHUNK)])


def kernel(positions, position_bias):
    tab = position_bias.reshape(-1)
    mesh = plsc.VectorSubcoreMesh(core_axis_name="c", subcore_axis_name="s")
    return pl.kernel(
        _body,
        out_type=jax.ShapeDtypeStruct((_BATCH, _HIST), jnp.float32),
        mesh=mesh,
        compiler_params=pltpu.CompilerParams(needs_layout_passes=False),
        scratch_types=[
            pltpu.VMEM((_TAB,), jnp.float32),
            pltpu.VMEM((_RCHUNK, _HIST), jnp.int32),
            pltpu.VMEM((_RCHUNK, _HIST), jnp.float32),
        ],
    )(positions.astype(jnp.int32), tab)
